# Initial kernel scaffold; baseline (speedup 1.0000x reference)
import jax
import jax.numpy as jnp
from jax.experimental import pallas as pl


def kernel(source_tensor, target_tensor):
    raise NotImplementedError("write your pallas kernel here")


# speedup vs baseline: 34.6802x; 34.6802x over previous
import functools

import jax
import jax.numpy as jnp
from jax import lax
from jax.experimental import pallas as pl
from jax.experimental.pallas import tpu as pltpu
from jax.experimental.pallas import tpu_sc as plsc

BINS = 256
LANES = 128


def _minmax_body(s_ref, t_ref, scale_ref, offs_ref):
    def params(x):
        mn = jnp.min(x)
        mx = jnp.max(x)
        width = (mx - mn) / BINS
        width = jnp.where(width == 0, jnp.float32(1.0), width)
        rw = 1.0 / width
        return rw, -mn * rw

    rw_s, b_s = params(s_ref[...])
    rw_t, b_t = params(t_ref[...])
    scale_ref[0, 0, :] = jnp.full((LANES,), rw_s, jnp.float32)
    scale_ref[0, 1, :] = jnp.full((LANES,), rw_t, jnp.float32)
    offs_ref[0, 0, :] = jnp.full((LANES,), b_s, jnp.float32)
    offs_ref[0, 1, :] = jnp.full((LANES,), b_t, jnp.float32)


def _minmax_tc(s3, t3):
    rows, sub, _ = s3.shape
    out_sd = jax.ShapeDtypeStruct((rows, 2, LANES), jnp.float32)
    return pl.pallas_call(
        _minmax_body,
        grid=(rows,),
        in_specs=[
            pl.BlockSpec((1, sub, LANES), lambda i: (i, 0, 0)),
            pl.BlockSpec((1, sub, LANES), lambda i: (i, 0, 0)),
        ],
        out_specs=[
            pl.BlockSpec((1, 2, LANES), lambda i: (i, 0, 0)),
            pl.BlockSpec((1, 2, LANES), lambda i: (i, 0, 0)),
        ],
        out_shape=[out_sd, out_sd],
    )(s3, t3)


CHUNK = 32768
UNROLL = 8


def _make_sc_binning(rows, row_len, rows_per_sub):
    nchunk = row_len // CHUNK
    mesh = plsc.VectorSubcoreMesh(core_axis_name="c", subcore_axis_name="s")

    @functools.partial(
        pl.kernel,
        mesh=mesh,
        compiler_params=pltpu.CompilerParams(needs_layout_passes=False),
        out_type=jax.ShapeDtypeStruct((2, rows, BINS), jnp.float32),
        scratch_types=[
            pltpu.VMEM((CHUNK,), jnp.float32),
            pltpu.VMEM((CHUNK,), jnp.float32),
            pltpu.VMEM((BINS,), jnp.float32),
            pltpu.VMEM((16,), jnp.float32),
            pltpu.VMEM((16,), jnp.float32),
            pltpu.SemaphoreType.DMA,
            pltpu.SemaphoreType.DMA,
        ],
    )
    def sc_binning(src, tgt, scale, offs, out, buf0, buf1, hist, a16, b16,
                   sem0, sem1):
        c = lax.axis_index("c")
        s = lax.axis_index("s")
        ones = jnp.ones((16,), jnp.float32)
        zeros = jnp.zeros((16,), jnp.float32)
        bufs = (buf0, buf1)
        sems = (sem0, sem1)

        def process(tref, tidx):
            def do_row(j, _):
                row = rows_per_sub * s + j
                pltpu.sync_copy(scale.at[row, tidx, pl.ds(0, 16)], a16)
                pltpu.sync_copy(offs.at[row, tidx, pl.ds(0, 16)], b16)
                a_v = a16[...]
                b_v = b16[...]
                for k in range(BINS // 16):
                    hist[pl.ds(16 * k, 16)] = zeros
                cp = pltpu.async_copy(tref.at[row, pl.ds(0, CHUNK)], buf0,
                                      sem0)
                for g in range(nchunk):
                    if g + 1 < nchunk:
                        nxt = (g + 1) % 2
                        cp_next = pltpu.async_copy(
                            tref.at[row, pl.ds((g + 1) * CHUNK, CHUNK)],
                            bufs[nxt], sems[nxt])
                    cp.wait()
                    buf = bufs[g % 2]

                    def body(i, _, buf=buf, a_v=a_v, b_v=b_v):
                        base = i * (16 * UNROLL)
                        for u in range(UNROLL):
                            x = buf[pl.ds(base + u * 16, 16)]
                            t = x * a_v + b_v
                            ix = jnp.minimum(t.astype(jnp.int32), BINS - 1)
                            plsc.addupdate_scatter(hist, [ix], ones)
                        return 0

                    lax.fori_loop(0, CHUNK // (16 * UNROLL), body, 0)
                    if g + 1 < nchunk:
                        cp = cp_next
                pltpu.sync_copy(hist, out.at[tidx, row])
                return 0

            lax.fori_loop(0, rows_per_sub, do_row, 0)

        @pl.when(c == 0)
        def _():
            process(src, 0)

        @pl.when(c == 1)
        def _():
            process(tgt, 1)

    return sc_binning


def _reduce_body(h_ref, out_ref, *, inv):
    h = h_ref[...]
    d = h[0] - h[1]
    out_ref[...] = jnp.reshape(jnp.sum(d * d) * inv, (1, 1))


def _reduce_tc(counts, rows):
    inv = 1.0 / (float(BINS) * float(BINS) * float(BINS) * float(rows))
    return pl.pallas_call(
        functools.partial(_reduce_body, inv=inv),
        out_shape=jax.ShapeDtypeStruct((1, 1), jnp.float32),
    )(counts)


def kernel(source_tensor, target_tensor):
    n, ch, h, w = source_tensor.shape
    rows = n * ch
    row_len = h * w
    s = source_tensor.reshape(rows, row_len)
    t = target_tensor.reshape(rows, row_len)
    s3 = s.reshape(rows, row_len // LANES, LANES)
    t3 = t.reshape(rows, row_len // LANES, LANES)
    scale, offs = _minmax_tc(s3, t3)
    rows_per_sub = rows // 16
    counts = _make_sc_binning(rows, row_len, rows_per_sub)(s, t, scale, offs)
    loss = _reduce_tc(counts, rows)
    return loss[0, 0]


# speedup vs baseline: 91.2647x; 2.6316x over previous
import functools

import jax
import jax.numpy as jnp
from jax import lax
from jax.experimental import pallas as pl
from jax.experimental.pallas import tpu as pltpu
from jax.experimental.pallas import tpu_sc as plsc

BINS = 256
LANES = 128


def _minmax_body(s_ref, t_ref, scale_ref, offs_ref):
    def params(x):
        mn = jnp.min(x)
        mx = jnp.max(x)
        width = (mx - mn) / BINS
        width = jnp.where(width == 0, jnp.float32(1.0), width)
        rw = 1.0 / width
        return rw, -mn * rw

    rw_s, b_s = params(s_ref[...])
    rw_t, b_t = params(t_ref[...])
    scale_ref[0, 0, :] = jnp.full((LANES,), rw_s, jnp.float32)
    scale_ref[0, 1, :] = jnp.full((LANES,), rw_t, jnp.float32)
    offs_ref[0, 0, :] = jnp.full((LANES,), b_s, jnp.float32)
    offs_ref[0, 1, :] = jnp.full((LANES,), b_t, jnp.float32)


def _minmax_tc(s3, t3):
    rows, sub, _ = s3.shape
    out_sd = jax.ShapeDtypeStruct((rows, 2, LANES), jnp.float32)
    return pl.pallas_call(
        _minmax_body,
        grid=(rows,),
        in_specs=[
            pl.BlockSpec((1, sub, LANES), lambda i: (i, 0, 0)),
            pl.BlockSpec((1, sub, LANES), lambda i: (i, 0, 0)),
        ],
        out_specs=[
            pl.BlockSpec((1, 2, LANES), lambda i: (i, 0, 0)),
            pl.BlockSpec((1, 2, LANES), lambda i: (i, 0, 0)),
        ],
        out_shape=[out_sd, out_sd],
    )(s3, t3)


CHUNK = 32768
UNROLL = 8


def _make_sc_binning(rows, row_len, rows_per_sub):
    nchunk = row_len // CHUNK
    mesh = plsc.VectorSubcoreMesh(core_axis_name="c", subcore_axis_name="s")

    @functools.partial(
        pl.kernel,
        mesh=mesh,
        compiler_params=pltpu.CompilerParams(needs_layout_passes=False),
        out_type=jax.ShapeDtypeStruct((2, rows, BINS), jnp.float32),
        scratch_types=[
            pltpu.VMEM((CHUNK,), jnp.float32),
            pltpu.VMEM((CHUNK,), jnp.float32),
            pltpu.VMEM((BINS,), jnp.float32),
            pltpu.VMEM((16,), jnp.float32),
            pltpu.VMEM((16,), jnp.float32),
            pltpu.SemaphoreType.DMA,
            pltpu.SemaphoreType.DMA,
        ],
    )
    def sc_binning(src, tgt, scale, offs, out, buf0, buf1, hist, a16, b16,
                   sem0, sem1):
        c = lax.axis_index("c")
        s = lax.axis_index("s")
        ones = jnp.ones((16,), jnp.float32)
        zeros = jnp.zeros((16,), jnp.float32)
        bufs = (buf0, buf1)
        sems = (sem0, sem1)

        def process(tref, tidx):
            def do_row(j, _):
                row = rows_per_sub * s + j
                pltpu.sync_copy(scale.at[row, tidx, pl.ds(0, 16)], a16)
                pltpu.sync_copy(offs.at[row, tidx, pl.ds(0, 16)], b16)
                a_v = a16[...]
                b_v = b16[...]
                for k in range(BINS // 16):
                    hist[pl.ds(16 * k, 16)] = zeros
                cp = pltpu.async_copy(tref.at[row, pl.ds(0, CHUNK)], buf0,
                                      sem0)
                for g in range(nchunk):
                    if g + 1 < nchunk:
                        nxt = (g + 1) % 2
                        cp_next = pltpu.async_copy(
                            tref.at[row, pl.ds((g + 1) * CHUNK, CHUNK)],
                            bufs[nxt], sems[nxt])
                    cp.wait()
                    buf = bufs[g % 2]

                    @plsc.parallel_loop(0, CHUNK // 16, 1, unroll=UNROLL)
                    def _(i, buf=buf, a_v=a_v, b_v=b_v):
                        x = buf[pl.ds(i * 16, 16)]
                        t = x * a_v + b_v
                        ix = jnp.minimum(t.astype(jnp.int32), BINS - 1)
                        plsc.addupdate_scatter(hist, [ix], ones)
                    if g + 1 < nchunk:
                        cp = cp_next
                pltpu.sync_copy(hist, out.at[tidx, row])
                return 0

            lax.fori_loop(0, rows_per_sub, do_row, 0)

        @pl.when(c == 0)
        def _():
            process(src, 0)

        @pl.when(c == 1)
        def _():
            process(tgt, 1)

    return sc_binning


def _reduce_body(h_ref, out_ref, *, inv):
    h = h_ref[...]
    d = h[0] - h[1]
    out_ref[...] = jnp.reshape(jnp.sum(d * d) * inv, (1, 1))


def _reduce_tc(counts, rows):
    inv = 1.0 / (float(BINS) * float(BINS) * float(BINS) * float(rows))
    return pl.pallas_call(
        functools.partial(_reduce_body, inv=inv),
        out_shape=jax.ShapeDtypeStruct((1, 1), jnp.float32),
    )(counts)


def kernel(source_tensor, target_tensor):
    n, ch, h, w = source_tensor.shape
    rows = n * ch
    row_len = h * w
    s = source_tensor.reshape(rows, row_len)
    t = target_tensor.reshape(rows, row_len)
    s3 = s.reshape(rows, row_len // LANES, LANES)
    t3 = t.reshape(rows, row_len // LANES, LANES)
    scale, offs = _minmax_tc(s3, t3)
    rows_per_sub = rows // 16
    counts = _make_sc_binning(rows, row_len, rows_per_sub)(s, t, scale, offs)
    loss = _reduce_tc(counts, rows)
    return loss[0, 0]


# speedup vs baseline: 165.0568x; 1.8086x over previous
import functools

import jax
import jax.numpy as jnp
from jax import lax
from jax.experimental import pallas as pl
from jax.experimental.pallas import tpu as pltpu
from jax.experimental.pallas import tpu_sc as plsc

BINS = 256
LANES = 128


def _minmax_body(s_ref, t_ref, scale_ref, offs_ref, *, nch):
    def params(x):
        mn = jnp.min(x)
        mx = jnp.max(x)
        width = (mx - mn) / BINS
        width = jnp.where(width == 0, jnp.float32(1.0), width)
        rw = 1.0 / width
        return rw, -mn * rw

    rw_s, b_s = params(s_ref[...])
    rw_t, b_t = params(t_ref[...])
    scale_ref[0, 0, :] = jnp.full((LANES,), rw_s, jnp.float32)
    scale_ref[0, 1, :] = jnp.full((LANES,), rw_t, jnp.float32)
    offs_ref[0, 0, :] = jnp.full((LANES,), b_s, jnp.float32)
    offs_ref[0, 1, :] = jnp.full((LANES,), b_t, jnp.float32)


def _minmax_tc(s4, t4):
    n, nch, h, w = s4.shape
    rows = n * nch
    out_sd = jax.ShapeDtypeStruct((rows, 2, LANES), jnp.float32)
    return pl.pallas_call(
        functools.partial(_minmax_body, nch=nch),
        grid=(rows,),
        in_specs=[
            pl.BlockSpec((1, 1, h, w), lambda i: (i // nch, i % nch, 0, 0)),
            pl.BlockSpec((1, 1, h, w), lambda i: (i // nch, i % nch, 0, 0)),
        ],
        out_specs=[
            pl.BlockSpec((1, 2, LANES), lambda i: (i, 0, 0)),
            pl.BlockSpec((1, 2, LANES), lambda i: (i, 0, 0)),
        ],
        out_shape=[out_sd, out_sd],
    )(s4, t4)


BLK_H = 64
UNROLL = 8


def _make_sc_binning(n, nch, h, w, rows_per_sub):
    rows = n * nch
    nblk = h // BLK_H
    groups_per_vec = w // 16
    vecs_per_blk = BLK_H * groups_per_vec
    mesh = plsc.VectorSubcoreMesh(core_axis_name="c", subcore_axis_name="s")

    @functools.partial(
        pl.kernel,
        mesh=mesh,
        compiler_params=pltpu.CompilerParams(
            needs_layout_passes=False, use_tc_tiling_on_sc=True),
        out_type=jax.ShapeDtypeStruct((2, rows, BINS), jnp.float32),
        scratch_types=[
            pltpu.VMEM((BLK_H, w), jnp.float32),
            pltpu.VMEM((BLK_H, w), jnp.float32),
            pltpu.VMEM((BINS,), jnp.float32),
            pltpu.VMEM((16,), jnp.float32),
            pltpu.VMEM((16,), jnp.float32),
            pltpu.SemaphoreType.DMA,
            pltpu.SemaphoreType.DMA,
        ],
    )
    def sc_binning(src, tgt, scale, offs, out, buf0, buf1, hist, a16, b16,
                   sem0, sem1):
        c = lax.axis_index("c")
        s = lax.axis_index("s")
        ones = jnp.ones((16,), jnp.float32)
        zeros = jnp.zeros((16,), jnp.float32)
        bufs = (buf0, buf1)
        sems = (sem0, sem1)

        def process(tref, tidx):
            def do_row(j, _):
                row = rows_per_sub * s + j
                ni = row // nch
                ci = row % nch
                pltpu.sync_copy(scale.at[row, tidx, pl.ds(0, 16)], a16)
                pltpu.sync_copy(offs.at[row, tidx, pl.ds(0, 16)], b16)
                a_v = a16[...]
                b_v = b16[...]
                for k in range(BINS // 16):
                    hist[pl.ds(16 * k, 16)] = zeros
                cp = pltpu.async_copy(
                    tref.at[ni, ci, pl.ds(0, BLK_H), :], buf0, sem0)
                for g in range(nblk):
                    if g + 1 < nblk:
                        nxt = (g + 1) % 2
                        cp_next = pltpu.async_copy(
                            tref.at[ni, ci, pl.ds((g + 1) * BLK_H, BLK_H), :],
                            bufs[nxt], sems[nxt])
                    cp.wait()
                    buf = bufs[g % 2]

                    @plsc.parallel_loop(0, vecs_per_blk, 1, unroll=UNROLL)
                    def _(i, buf=buf, a_v=a_v, b_v=b_v):
                        r = i // groups_per_vec
                        col = (i % groups_per_vec) * 16
                        x = buf[r, pl.ds(col, 16)]
                        t = x * a_v + b_v
                        ix = jnp.minimum(t.astype(jnp.int32), BINS - 1)
                        plsc.addupdate_scatter(hist, [ix], ones)

                    if g + 1 < nblk:
                        cp = cp_next
                pltpu.sync_copy(hist, out.at[tidx, row])
                return 0

            lax.fori_loop(0, rows_per_sub, do_row, 0)

        @pl.when(c == 0)
        def _():
            process(src, 0)

        @pl.when(c == 1)
        def _():
            process(tgt, 1)

    return sc_binning


def _reduce_body(h_ref, out_ref, *, inv):
    h = h_ref[...]
    d = h[0] - h[1]
    out_ref[...] = jnp.reshape(jnp.sum(d * d) * inv, (1, 1))


def _reduce_tc(counts, rows):
    inv = 1.0 / (float(BINS) * float(BINS) * float(BINS) * float(rows))
    return pl.pallas_call(
        functools.partial(_reduce_body, inv=inv),
        out_shape=jax.ShapeDtypeStruct((1, 1), jnp.float32),
    )(counts)


def kernel(source_tensor, target_tensor):
    n, nch, h, w = source_tensor.shape
    rows = n * nch
    scale, offs = _minmax_tc(source_tensor, target_tensor)
    rows_per_sub = rows // 16
    counts = _make_sc_binning(n, nch, h, w, rows_per_sub)(
        source_tensor, target_tensor, scale, offs)
    loss = _reduce_tc(counts, rows)
    return loss[0, 0]


# speedup vs baseline: 165.3128x; 1.0016x over previous
import functools

import jax
import jax.numpy as jnp
from jax import lax
from jax.experimental import pallas as pl
from jax.experimental.pallas import tpu as pltpu
from jax.experimental.pallas import tpu_sc as plsc

BINS = 256
LANES = 128


def _minmax_body(s_ref, t_ref, scale_ref, offs_ref, *, nch):
    def params(x):
        mn = jnp.min(x)
        mx = jnp.max(x)
        width = (mx - mn) / BINS
        width = jnp.where(width == 0, jnp.float32(1.0), width)
        rw = 1.0 / width
        return rw, -mn * rw

    rw_s, b_s = params(s_ref[...])
    rw_t, b_t = params(t_ref[...])
    scale_ref[0, 0, :] = jnp.full((LANES,), rw_s, jnp.float32)
    scale_ref[0, 1, :] = jnp.full((LANES,), rw_t, jnp.float32)
    offs_ref[0, 0, :] = jnp.full((LANES,), b_s, jnp.float32)
    offs_ref[0, 1, :] = jnp.full((LANES,), b_t, jnp.float32)


def _minmax_tc(s4, t4):
    n, nch, h, w = s4.shape
    rows = n * nch
    out_sd = jax.ShapeDtypeStruct((rows, 2, LANES), jnp.float32)
    return pl.pallas_call(
        functools.partial(_minmax_body, nch=nch),
        grid=(rows,),
        in_specs=[
            pl.BlockSpec((1, 1, h, w), lambda i: (i // nch, i % nch, 0, 0)),
            pl.BlockSpec((1, 1, h, w), lambda i: (i // nch, i % nch, 0, 0)),
        ],
        out_specs=[
            pl.BlockSpec((1, 2, LANES), lambda i: (i, 0, 0)),
            pl.BlockSpec((1, 2, LANES), lambda i: (i, 0, 0)),
        ],
        out_shape=[out_sd, out_sd],
    )(s4, t4)


BLK_H = 64
UNROLL = 8


def _make_sc_binning(n, nch, h, w, rows_per_sub):
    rows = n * nch
    nblk = h // BLK_H
    groups_per_vec = w // 16
    vecs_per_blk = BLK_H * groups_per_vec
    mesh = plsc.VectorSubcoreMesh(core_axis_name="c", subcore_axis_name="s")

    @functools.partial(
        pl.kernel,
        mesh=mesh,
        compiler_params=pltpu.CompilerParams(
            needs_layout_passes=False, use_tc_tiling_on_sc=True),
        out_type=jax.ShapeDtypeStruct((2, rows, BINS), jnp.float32),
        scratch_types=[
            pltpu.VMEM((BLK_H, w), jnp.float32),
            pltpu.VMEM((BLK_H, w), jnp.float32),
            pltpu.VMEM((BINS + 16,), jnp.float32),
            pltpu.VMEM((16,), jnp.float32),
            pltpu.VMEM((16,), jnp.float32),
            pltpu.SemaphoreType.DMA,
            pltpu.SemaphoreType.DMA,
        ],
    )
    def sc_binning(src, tgt, scale, offs, out, buf0, buf1, hist, a16, b16,
                   sem0, sem1):
        c = lax.axis_index("c")
        s = lax.axis_index("s")
        ones = jnp.ones((16,), jnp.float32)
        zeros = jnp.zeros((16,), jnp.float32)
        lane0 = lax.iota(jnp.int32, 16) == 0
        full255 = jnp.full((16,), BINS - 1, jnp.int32)
        bufs = (buf0, buf1)
        sems = (sem0, sem1)

        def process(tref, tidx):
            def do_row(j, _):
                row = rows_per_sub * s + j
                ni = row // nch
                ci = row % nch
                pltpu.sync_copy(scale.at[row, tidx, pl.ds(0, 16)], a16)
                pltpu.sync_copy(offs.at[row, tidx, pl.ds(0, 16)], b16)
                a_v = a16[...]
                b_v = b16[...]
                for k in range((BINS + 16) // 16):
                    hist[pl.ds(16 * k, 16)] = zeros
                cp = pltpu.async_copy(
                    tref.at[ni, ci, pl.ds(0, BLK_H), :], buf0, sem0)
                for g in range(nblk):
                    if g + 1 < nblk:
                        nxt = (g + 1) % 2
                        cp_next = pltpu.async_copy(
                            tref.at[ni, ci, pl.ds((g + 1) * BLK_H, BLK_H), :],
                            bufs[nxt], sems[nxt])
                    cp.wait()
                    buf = bufs[g % 2]

                    @plsc.parallel_loop(0, vecs_per_blk, 1, unroll=UNROLL)
                    def _(i, buf=buf, a_v=a_v, b_v=b_v):
                        r = i // groups_per_vec
                        col = (i % groups_per_vec) * 16
                        x = buf[r, pl.ds(col, 16)]
                        t = x * a_v + b_v
                        plsc.addupdate_scatter(hist, [t.astype(jnp.int32)],
                                               ones)

                    if g + 1 < nblk:
                        cp = cp_next
                overflow = hist[pl.ds(BINS, 16)]
                plsc.addupdate_scatter(hist, [full255], overflow, mask=lane0)
                pltpu.sync_copy(hist.at[pl.ds(0, BINS)], out.at[tidx, row])
                return 0

            lax.fori_loop(0, rows_per_sub, do_row, 0)

        @pl.when(c == 0)
        def _():
            process(src, 0)

        @pl.when(c == 1)
        def _():
            process(tgt, 1)

    return sc_binning


def _reduce_body(h_ref, out_ref, *, inv):
    h = h_ref[...]
    d = h[0] - h[1]
    out_ref[...] = jnp.reshape(jnp.sum(d * d) * inv, (1, 1))


def _reduce_tc(counts, rows):
    inv = 1.0 / (float(BINS) * float(BINS) * float(BINS) * float(rows))
    return pl.pallas_call(
        functools.partial(_reduce_body, inv=inv),
        out_shape=jax.ShapeDtypeStruct((1, 1), jnp.float32),
    )(counts)


def kernel(source_tensor, target_tensor):
    n, nch, h, w = source_tensor.shape
    rows = n * nch
    scale, offs = _minmax_tc(source_tensor, target_tensor)
    rows_per_sub = rows // 16
    counts = _make_sc_binning(n, nch, h, w, rows_per_sub)(
        source_tensor, target_tensor, scale, offs)
    loss = _reduce_tc(counts, rows)
    return loss[0, 0]
